# baseline (speedup 1.0000x reference)
import jax
import jax.numpy as jnp
from jax import lax
from jax.experimental import pallas as pl
from jax.experimental.pallas import tpu as pltpu
from jax.experimental.pallas import tpu_sc as plsc

NC = 2
NS = 16
L = 16

F_OUT = 128
CHUNK = 400
CHUNK_S = 256
BM_E = 2048
NEG_SLOPE = 0.2


def _mm_body(x_ref, wl_ref, wr_ref, xl_ref, xr_ref):
    xb = x_ref[...]
    xl_ref[...] = jnp.dot(xb, wl_ref[...], preferred_element_type=jnp.float32)
    xr_ref[...] = jnp.dot(xb, wr_ref[...], preferred_element_type=jnp.float32)


def _project(x, W_l, W_r):
    n, f_in = x.shape
    bm = 1000
    return pl.pallas_call(
        _mm_body,
        grid=(n // bm,),
        in_specs=[
            pl.BlockSpec((bm, f_in), lambda i: (i, 0)),
            pl.BlockSpec((f_in, F_OUT), lambda i: (0, 0)),
            pl.BlockSpec((f_in, F_OUT), lambda i: (0, 0)),
        ],
        out_specs=[
            pl.BlockSpec((bm, F_OUT), lambda i: (i, 0)),
            pl.BlockSpec((bm, F_OUT), lambda i: (i, 0)),
        ],
        out_shape=[
            jax.ShapeDtypeStruct((n, F_OUT), jnp.float32),
            jax.ShapeDtypeStruct((n, F_OUT), jnp.float32),
        ],
    )(x, W_l, W_r)


def _logits_body(xlr_ref, xrr_ref, ea_ref, we_ref, att_ref,
                 lg_ref, bmax_ref, max_acc):
    i = pl.program_id(0)
    nb = pl.num_programs(0)
    ea = ea_ref[...].reshape(BM_E, 1)
    m = xlr_ref[...] + xrr_ref[...] + ea * we_ref[...]
    m = jnp.maximum(m, m * NEG_SLOPE)
    lg = jnp.dot(m, att_ref[...], preferred_element_type=jnp.float32)
    lg_ref[...] = lg.reshape(BM_E)

    @pl.when(i == 0)
    def _():
        max_acc[...] = jnp.full_like(max_acc, -jnp.inf)

    max_acc[...] = jnp.maximum(max_acc[...], jnp.max(lg))

    @pl.when(i == nb - 1)
    def _():
        bmax_ref[...] = max_acc[...]


def _edge_logits(xlr, xrr, ea, we2, att2):
    e = xlr.shape[0]
    nb = e // BM_E
    return pl.pallas_call(
        _logits_body,
        grid=(nb,),
        in_specs=[
            pl.BlockSpec((BM_E, F_OUT), lambda i: (i, 0)),
            pl.BlockSpec((BM_E, F_OUT), lambda i: (i, 0)),
            pl.BlockSpec((BM_E,), lambda i: (i,)),
            pl.BlockSpec((1, F_OUT), lambda i: (0, 0)),
            pl.BlockSpec((F_OUT, 1), lambda i: (0, 0)),
        ],
        out_specs=[
            pl.BlockSpec((BM_E,), lambda i: (i,)),
            pl.BlockSpec((1, F_OUT), lambda i: (0, 0)),
        ],
        out_shape=[
            jax.ShapeDtypeStruct((e,), jnp.float32),
            jax.ShapeDtypeStruct((1, F_OUT), jnp.float32),
        ],
        scratch_shapes=[pltpu.VMEM((1, F_OUT), jnp.float32)],
    )(xlr, xrr, ea, we2, att2)


def _scale_body(lg_ref, xlr_ref, bmax_ref, ex_ref, scaled_ref):
    gmax = jnp.max(bmax_ref[...])
    ex = jnp.exp(lg_ref[...] - gmax)
    ex_ref[...] = ex
    scaled_ref[...] = xlr_ref[...] * ex.reshape(BM_E, 1)


def _edge_scale(lg, xlr, bmax):
    e = xlr.shape[0]
    nb = e // BM_E
    return pl.pallas_call(
        _scale_body,
        grid=(nb,),
        in_specs=[
            pl.BlockSpec((BM_E,), lambda i: (i,)),
            pl.BlockSpec((BM_E, F_OUT), lambda i: (i, 0)),
            pl.BlockSpec((1, F_OUT), lambda i: (0, 0)),
        ],
        out_specs=[
            pl.BlockSpec((BM_E,), lambda i: (i,)),
            pl.BlockSpec((BM_E, F_OUT), lambda i: (i, 0)),
        ],
        out_shape=[
            jax.ShapeDtypeStruct((e,), jnp.float32),
            jax.ShapeDtypeStruct((e, F_OUT), jnp.float32),
        ],
    )(lg, xlr, bmax)


def _finish_body(agg_ref, d0_ref, d1_ref, batch_ref, bgat_ref, wlin_ref,
                 blin_ref, out_ref, pooled_acc, cnt_acc):
    i = pl.program_id(0)
    nb = pl.num_programs(0)
    num_graphs = out_ref.shape[0]

    @pl.when(i == 0)
    def _():
        pooled_acc[...] = jnp.zeros_like(pooled_acc)
        cnt_acc[...] = jnp.zeros_like(cnt_acc)

    den = d0_ref[...] + d1_ref[...] + 1e-16
    h = (agg_ref[0] + agg_ref[1]) / den + bgat_ref[...]
    h = jnp.maximum(h, 0.0)

    gids = lax.broadcasted_iota(jnp.int32, (1, num_graphs), 1)
    oneh = (batch_ref[...] == gids).astype(jnp.float32)
    dims = (((0,), (0,)), ((), ()))
    pooled_acc[...] += lax.dot_general(oneh, h, dims,
                                       preferred_element_type=jnp.float32)
    ones = jnp.ones(h.shape, jnp.float32)
    cnt_acc[...] += lax.dot_general(oneh, ones, dims,
                                    preferred_element_type=jnp.float32)

    @pl.when(i == nb - 1)
    def _():
        pooled = pooled_acc[...] / jnp.maximum(cnt_acc[...], 1.0)
        out_ref[...] = jnp.dot(pooled, wlin_ref[...],
                               preferred_element_type=jnp.float32) + blin_ref[...]


def _finish(agg2, d0, d1, batch2, bgat2, W_lin, blin2, num_graphs, num_classes):
    n = agg2.shape[1]
    bm = 1000
    return pl.pallas_call(
        _finish_body,
        grid=(n // bm,),
        in_specs=[
            pl.BlockSpec((NC, bm, F_OUT), lambda i: (0, i, 0)),
            pl.BlockSpec((bm, 1), lambda i: (i, 0)),
            pl.BlockSpec((bm, 1), lambda i: (i, 0)),
            pl.BlockSpec((bm, 1), lambda i: (i, 0)),
            pl.BlockSpec((1, F_OUT), lambda i: (0, 0)),
            pl.BlockSpec((F_OUT, num_classes), lambda i: (0, 0)),
            pl.BlockSpec((1, num_classes), lambda i: (0, 0)),
        ],
        out_specs=pl.BlockSpec((num_graphs, num_classes), lambda i: (0, 0)),
        out_shape=jax.ShapeDtypeStruct((num_graphs, num_classes), jnp.float32),
        scratch_shapes=[
            pltpu.VMEM((num_graphs, F_OUT), jnp.float32),
            pltpu.VMEM((num_graphs, F_OUT), jnp.float32),
        ],
    )(agg2, d0, d1, batch2, bgat2, W_lin, blin2)


def _gather_body(xl_hbm, xr_hbm, src_hbm, dst_hbm, xlr_hbm, xrr_hbm,
                 src_v, dst_v, xl_rows, xr_rows, sem):
    c = lax.axis_index("c")
    s = lax.axis_index("s")
    wid = c * NS + s
    e_total = src_hbm.shape[0]
    per_w = e_total // (NC * NS)
    base_w = wid * per_w

    def piece(base, sz):
        pltpu.sync_copy(src_hbm.at[pl.ds(base, sz)], src_v.at[pl.ds(0, sz)])
        pltpu.sync_copy(dst_hbm.at[pl.ds(base, sz)], dst_v.at[pl.ds(0, sz)])
        a = pltpu.async_copy(xl_hbm.at[src_v.at[pl.ds(0, sz)]],
                             xl_rows.at[pl.ds(0, sz)], sem)
        b = pltpu.async_copy(xr_hbm.at[dst_v.at[pl.ds(0, sz)]],
                             xr_rows.at[pl.ds(0, sz)], sem)
        a.wait()
        b.wait()
        pltpu.sync_copy(xl_rows.at[pl.ds(0, sz)], xlr_hbm.at[pl.ds(base, sz)])
        pltpu.sync_copy(xr_rows.at[pl.ds(0, sz)], xrr_hbm.at[pl.ds(base, sz)])

    n_full = per_w // CHUNK

    def chunk_body(ci, cr):
        piece(base_w + ci * CHUNK, CHUNK)
        return cr

    lax.fori_loop(0, n_full, chunk_body, jnp.int32(0))
    tail = per_w - n_full * CHUNK
    if tail:
        piece(base_w + n_full * CHUNK, tail)


def _scatter_body(scaled_hbm, ex_hbm, dst_hbm,
                  agg_hbm, den_hbm,
                  dst_v, ex_v, rows_v, zbuf,
                  agg_sp, den_sp, sem):
    c = lax.axis_index("c")
    s = lax.axis_index("s")
    wid = c * NS + s
    e_total = dst_hbm.shape[0]
    n = agg_hbm.shape[1]
    per_w = e_total // (NC * NS)
    base_w = wid * per_w

    n_t0 = (n // NS) // 8 * 8
    n_last = n - n_t0 * (NS - 1)
    n_pad_rows = agg_sp.shape[0] - n
    nd_pad = den_sp.shape[0]
    nd_tile = nd_pad // NS

    zv = jnp.zeros((L,), jnp.float32)

    def zrow(r, cr):
        def zcol(k, ck):
            rows_v[r, pl.ds(k * L, L)] = zv
            return ck
        return lax.fori_loop(0, F_OUT // L, zcol, cr)

    lax.fori_loop(0, CHUNK_S, zrow, jnp.int32(0))

    def zflat(k, ck):
        zbuf[pl.ds(k * L, L)] = zv
        return ck

    lax.fori_loop(0, zbuf.shape[0] // L, zflat, jnp.int32(0))

    def _zero_agg_rows(r0, total):
        done = 0
        while total - done > 0:
            sz = min(CHUNK_S, total - done)
            pltpu.sync_copy(rows_v.at[pl.ds(0, sz)],
                            agg_sp.at[pl.ds(r0 + done, sz)])
            done += sz

    @pl.when(s < NS - 1)
    def _():
        _zero_agg_rows(s * n_t0, n_t0)

    @pl.when(s == NS - 1)
    def _():
        _zero_agg_rows((NS - 1) * n_t0, n_last + n_pad_rows)

    pltpu.sync_copy(zbuf.at[pl.ds(0, nd_tile)],
                    den_sp.at[pl.ds(s * nd_tile, nd_tile)])

    plsc.subcore_barrier()

    def piece(base, sz):
        pltpu.sync_copy(dst_hbm.at[pl.ds(base, sz)], dst_v.at[pl.ds(0, sz)])
        a = pltpu.async_copy(ex_hbm.at[pl.ds(base, sz)],
                             ex_v.at[pl.ds(0, sz)], sem)
        b = pltpu.async_copy(scaled_hbm.at[pl.ds(base, sz)],
                             rows_v.at[pl.ds(0, sz)], sem)
        a.wait()
        b.wait()
        pltpu.sync_copy(ex_v.at[pl.ds(0, sz)],
                        den_sp.at[dst_v.at[pl.ds(0, sz)]], add=True)
        pltpu.sync_copy(rows_v.at[pl.ds(0, sz)],
                        agg_sp.at[dst_v.at[pl.ds(0, sz)]], add=True)

    n_full = per_w // CHUNK_S

    def chunk_body(ci, cr):
        piece(base_w + ci * CHUNK_S, CHUNK_S)
        return cr

    lax.fori_loop(0, n_full, chunk_body, jnp.int32(0))
    tail = per_w - n_full * CHUNK_S
    if tail:
        piece(base_w + n_full * CHUNK_S, tail)

    plsc.subcore_barrier()

    def _drain_agg_rows(r0, total):
        done = 0
        while done < total:
            sz = min(CHUNK_S, total - done)
            pltpu.sync_copy(agg_sp.at[pl.ds(r0 + done, sz)],
                            rows_v.at[pl.ds(0, sz)])
            pltpu.sync_copy(rows_v.at[pl.ds(0, sz)],
                            agg_hbm.at[c, pl.ds(r0 + done, sz)])
            done += sz

    @pl.when(s < NS - 1)
    def _():
        _drain_agg_rows(s * n_t0, n_t0)

    @pl.when(s == NS - 1)
    def _():
        _drain_agg_rows((NS - 1) * n_t0, n_last)

    pltpu.sync_copy(den_sp.at[pl.ds(s * nd_tile, nd_tile)],
                    zbuf.at[pl.ds(0, nd_tile)])
    pltpu.sync_copy(zbuf.at[pl.ds(0, nd_tile)],
                    den_hbm.at[pl.ds(c * nd_pad + s * nd_tile, nd_tile)])


def kernel(x, edge_index, edge_attr, batch, W_l, W_r, att, W_e, b_gat,
           W_lin, b_lin):
    n = x.shape[0]
    e = edge_index.shape[1]
    num_graphs = 64
    num_classes = W_lin.shape[1]
    nd_pad = ((n + NS * 8 - 1) // (NS * 8)) * (NS * 8)
    ep = ((e + BM_E - 1) // BM_E) * BM_E

    x_l, x_r = _project(x, W_l, W_r)

    pad = ep - e
    src = jnp.concatenate([edge_index[0], jnp.zeros((pad,), jnp.int32)])
    dst_g = jnp.concatenate([edge_index[1], jnp.zeros((pad,), jnp.int32)])
    dst_s = jnp.concatenate([edge_index[1], jnp.full((pad,), n, jnp.int32)])
    ea = jnp.concatenate([edge_attr[:, 0], jnp.zeros((pad,), jnp.float32)])

    mesh = plsc.VectorSubcoreMesh(core_axis_name="c", subcore_axis_name="s",
                                  num_cores=NC, num_subcores=NS)
    sc_params = pltpu.CompilerParams(needs_layout_passes=False)

    gather_fn = pl.kernel(
        _gather_body,
        out_type=[
            jax.ShapeDtypeStruct((ep, F_OUT), jnp.float32),
            jax.ShapeDtypeStruct((ep, F_OUT), jnp.float32),
        ],
        mesh=mesh,
        compiler_params=sc_params,
        scratch_types=[
            pltpu.VMEM((CHUNK,), jnp.int32),
            pltpu.VMEM((CHUNK,), jnp.int32),
            pltpu.VMEM((CHUNK, F_OUT), jnp.float32),
            pltpu.VMEM((CHUNK, F_OUT), jnp.float32),
            pltpu.SemaphoreType.DMA,
        ],
    )
    xlr, xrr = gather_fn(x_l, x_r, src, dst_g)

    we2 = W_e
    att2 = att[:, None]
    lg, bmax = _edge_logits(xlr, xrr, ea, we2, att2)
    ex, scaled = _edge_scale(lg, xlr, bmax)

    scatter_fn = pl.kernel(
        _scatter_body,
        out_type=[
            jax.ShapeDtypeStruct((NC, n, F_OUT), jnp.float32),
            jax.ShapeDtypeStruct((NC * nd_pad,), jnp.float32),
        ],
        mesh=mesh,
        compiler_params=sc_params,
        scratch_types=[
            pltpu.VMEM((CHUNK_S,), jnp.int32),
            pltpu.VMEM((CHUNK_S,), jnp.float32),
            pltpu.VMEM((CHUNK_S, F_OUT), jnp.float32),
            pltpu.VMEM((640,), jnp.float32),
            pltpu.VMEM_SHARED((n + 8, F_OUT), jnp.float32),
            pltpu.VMEM_SHARED((nd_pad,), jnp.float32),
            pltpu.SemaphoreType.DMA,
        ],
    )
    agg2, den2 = scatter_fn(scaled, ex, dst_s)

    d0 = den2[:n, None]
    d1 = den2[nd_pad:nd_pad + n, None]
    batch2 = batch[:, None]
    bgat2 = b_gat[None, :]
    blin2 = b_lin[None, :]
    return _finish(agg2, d0, d1, batch2, bgat2, W_lin, blin2,
                   num_graphs, num_classes)


# speedup vs baseline: 1.3358x; 1.3358x over previous
import jax
import jax.numpy as jnp
from jax import lax
from jax.experimental import pallas as pl
from jax.experimental.pallas import tpu as pltpu
from jax.experimental.pallas import tpu_sc as plsc

NC = 2
NS = 16
L = 16

F_OUT = 128
CHUNK = 400
CHUNK_S = 256
BM_E = 2048
NEG_SLOPE = 0.2


def _mm_body(x_ref, wl_ref, wr_ref, xl_ref, xr_ref):
    xb = x_ref[...]
    xl_ref[...] = jnp.dot(xb, wl_ref[...], preferred_element_type=jnp.float32)
    xr_ref[...] = jnp.dot(xb, wr_ref[...], preferred_element_type=jnp.float32)


def _project(x, W_l, W_r):
    n, f_in = x.shape
    bm = 1000
    return pl.pallas_call(
        _mm_body,
        grid=(n // bm,),
        in_specs=[
            pl.BlockSpec((bm, f_in), lambda i: (i, 0)),
            pl.BlockSpec((f_in, F_OUT), lambda i: (0, 0)),
            pl.BlockSpec((f_in, F_OUT), lambda i: (0, 0)),
        ],
        out_specs=[
            pl.BlockSpec((bm, F_OUT), lambda i: (i, 0)),
            pl.BlockSpec((bm, F_OUT), lambda i: (i, 0)),
        ],
        out_shape=[
            jax.ShapeDtypeStruct((n, F_OUT), jnp.float32),
            jax.ShapeDtypeStruct((n, F_OUT), jnp.float32),
        ],
    )(x, W_l, W_r)


def _logits_body(xlr_ref, xrr_ref, ea_ref, we_ref, att_ref,
                 lg_ref, bmax_ref, max_acc):
    i = pl.program_id(0)
    nb = pl.num_programs(0)
    ea = ea_ref[...].reshape(BM_E, 1)
    m = xlr_ref[...] + xrr_ref[...] + ea * we_ref[...]
    m = jnp.maximum(m, m * NEG_SLOPE)
    lg = jnp.dot(m, att_ref[...], preferred_element_type=jnp.float32)
    lg_ref[...] = lg.reshape(BM_E)

    @pl.when(i == 0)
    def _():
        max_acc[...] = jnp.full_like(max_acc, -jnp.inf)

    max_acc[...] = jnp.maximum(max_acc[...], jnp.max(lg))

    @pl.when(i == nb - 1)
    def _():
        bmax_ref[...] = max_acc[...]


def _edge_logits(xlr, xrr, ea, we2, att2):
    e = xlr.shape[0]
    nb = e // BM_E
    return pl.pallas_call(
        _logits_body,
        grid=(nb,),
        in_specs=[
            pl.BlockSpec((BM_E, F_OUT), lambda i: (i, 0)),
            pl.BlockSpec((BM_E, F_OUT), lambda i: (i, 0)),
            pl.BlockSpec((BM_E,), lambda i: (i,)),
            pl.BlockSpec((1, F_OUT), lambda i: (0, 0)),
            pl.BlockSpec((F_OUT, 1), lambda i: (0, 0)),
        ],
        out_specs=[
            pl.BlockSpec((BM_E,), lambda i: (i,)),
            pl.BlockSpec((1, F_OUT), lambda i: (0, 0)),
        ],
        out_shape=[
            jax.ShapeDtypeStruct((e,), jnp.float32),
            jax.ShapeDtypeStruct((1, F_OUT), jnp.float32),
        ],
        scratch_shapes=[pltpu.VMEM((1, F_OUT), jnp.float32)],
    )(xlr, xrr, ea, we2, att2)


def _scale_body(lg_ref, xlr_ref, bmax_ref, ex_ref, scaled_ref):
    gmax = jnp.max(bmax_ref[...])
    ex = jnp.exp(lg_ref[...] - gmax)
    ex_ref[...] = ex
    scaled_ref[...] = xlr_ref[...] * ex.reshape(BM_E, 1)


def _edge_scale(lg, xlr, bmax):
    e = xlr.shape[0]
    nb = e // BM_E
    return pl.pallas_call(
        _scale_body,
        grid=(nb,),
        in_specs=[
            pl.BlockSpec((BM_E,), lambda i: (i,)),
            pl.BlockSpec((BM_E, F_OUT), lambda i: (i, 0)),
            pl.BlockSpec((1, F_OUT), lambda i: (0, 0)),
        ],
        out_specs=[
            pl.BlockSpec((BM_E,), lambda i: (i,)),
            pl.BlockSpec((BM_E, F_OUT), lambda i: (i, 0)),
        ],
        out_shape=[
            jax.ShapeDtypeStruct((e,), jnp.float32),
            jax.ShapeDtypeStruct((e, F_OUT), jnp.float32),
        ],
    )(lg, xlr, bmax)


def _finish_body(agg_ref, d0_ref, d1_ref, batch_ref, bgat_ref, wlin_ref,
                 blin_ref, out_ref, pooled_acc, cnt_acc):
    i = pl.program_id(0)
    nb = pl.num_programs(0)
    num_graphs = out_ref.shape[0]

    @pl.when(i == 0)
    def _():
        pooled_acc[...] = jnp.zeros_like(pooled_acc)
        cnt_acc[...] = jnp.zeros_like(cnt_acc)

    den = d0_ref[...] + d1_ref[...] + 1e-16
    h = (agg_ref[0] + agg_ref[1]) / den + bgat_ref[...]
    h = jnp.maximum(h, 0.0)

    gids = lax.broadcasted_iota(jnp.int32, (1, num_graphs), 1)
    oneh = (batch_ref[...] == gids).astype(jnp.float32)
    dims = (((0,), (0,)), ((), ()))
    pooled_acc[...] += lax.dot_general(oneh, h, dims,
                                       preferred_element_type=jnp.float32)
    ones = jnp.ones(h.shape, jnp.float32)
    cnt_acc[...] += lax.dot_general(oneh, ones, dims,
                                    preferred_element_type=jnp.float32)

    @pl.when(i == nb - 1)
    def _():
        pooled = pooled_acc[...] / jnp.maximum(cnt_acc[...], 1.0)
        out_ref[...] = jnp.dot(pooled, wlin_ref[...],
                               preferred_element_type=jnp.float32) + blin_ref[...]


def _finish(agg2, d0, d1, batch2, bgat2, W_lin, blin2, num_graphs, num_classes):
    n = agg2.shape[1]
    bm = 1000
    return pl.pallas_call(
        _finish_body,
        grid=(n // bm,),
        in_specs=[
            pl.BlockSpec((NC, bm, F_OUT), lambda i: (0, i, 0)),
            pl.BlockSpec((bm, 1), lambda i: (i, 0)),
            pl.BlockSpec((bm, 1), lambda i: (i, 0)),
            pl.BlockSpec((bm, 1), lambda i: (i, 0)),
            pl.BlockSpec((1, F_OUT), lambda i: (0, 0)),
            pl.BlockSpec((F_OUT, num_classes), lambda i: (0, 0)),
            pl.BlockSpec((1, num_classes), lambda i: (0, 0)),
        ],
        out_specs=pl.BlockSpec((num_graphs, num_classes), lambda i: (0, 0)),
        out_shape=jax.ShapeDtypeStruct((num_graphs, num_classes), jnp.float32),
        scratch_shapes=[
            pltpu.VMEM((num_graphs, F_OUT), jnp.float32),
            pltpu.VMEM((num_graphs, F_OUT), jnp.float32),
        ],
    )(agg2, d0, d1, batch2, bgat2, W_lin, blin2)


def _gather_body(xl_hbm, xr_hbm, src_hbm, dst_hbm, xlr_hbm, xrr_hbm,
                 src_v, dst_v, xl_rows, xr_rows, sem):
    c = lax.axis_index("c")
    s = lax.axis_index("s")
    wid = c * NS + s
    e_total = src_hbm.shape[0]
    per_w = e_total // (NC * NS)
    base_w = wid * per_w

    def piece(base, sz):
        pltpu.sync_copy(src_hbm.at[pl.ds(base, sz)], src_v.at[pl.ds(0, sz)])
        pltpu.sync_copy(dst_hbm.at[pl.ds(base, sz)], dst_v.at[pl.ds(0, sz)])
        a = pltpu.async_copy(xl_hbm.at[src_v.at[pl.ds(0, sz)]],
                             xl_rows.at[pl.ds(0, sz)], sem)
        b = pltpu.async_copy(xr_hbm.at[dst_v.at[pl.ds(0, sz)]],
                             xr_rows.at[pl.ds(0, sz)], sem)
        a.wait()
        b.wait()
        pltpu.sync_copy(xl_rows.at[pl.ds(0, sz)], xlr_hbm.at[pl.ds(base, sz)])
        pltpu.sync_copy(xr_rows.at[pl.ds(0, sz)], xrr_hbm.at[pl.ds(base, sz)])

    n_full = per_w // CHUNK

    def chunk_body(ci, cr):
        piece(base_w + ci * CHUNK, CHUNK)
        return cr

    lax.fori_loop(0, n_full, chunk_body, jnp.int32(0))
    tail = per_w - n_full * CHUNK
    if tail:
        piece(base_w + n_full * CHUNK, tail)


def _scatter_body(scaled_hbm, ex_hbm, dst_hbm,
                  agg_hbm, den_hbm,
                  dst_v, ex_v, rows_v, zbuf,
                  agg_sp, den_sp, sem):
    c = lax.axis_index("c")
    s = lax.axis_index("s")
    wid = c * NS + s
    e_total = dst_hbm.shape[0]
    n = agg_hbm.shape[1]
    per_w = e_total // (NC * NS)
    base_w = wid * per_w

    n_t0 = (n // NS) // 8 * 8
    n_last = n - n_t0 * (NS - 1)
    n_pad_rows = agg_sp.shape[0] - n
    nd_pad = den_sp.shape[0]
    nd_tile = nd_pad // NS

    zv = jnp.zeros((L,), jnp.float32)

    def zrow(r, cr):
        def zcol(k, ck):
            rows_v[r, pl.ds(k * L, L)] = zv
            return ck
        return lax.fori_loop(0, F_OUT // L, zcol, cr)

    lax.fori_loop(0, CHUNK_S, zrow, jnp.int32(0))

    def zflat(k, ck):
        zbuf[pl.ds(k * L, L)] = zv
        return ck

    lax.fori_loop(0, zbuf.shape[0] // L, zflat, jnp.int32(0))

    def _zero_agg_rows(r0, total):
        done = 0
        while total - done > 0:
            sz = min(CHUNK_S, total - done)
            pltpu.sync_copy(rows_v.at[pl.ds(0, sz)],
                            agg_sp.at[pl.ds(r0 + done, sz)])
            done += sz

    @pl.when(s < NS - 1)
    def _():
        _zero_agg_rows(s * n_t0, n_t0)

    @pl.when(s == NS - 1)
    def _():
        _zero_agg_rows((NS - 1) * n_t0, n_last + n_pad_rows)

    pltpu.sync_copy(zbuf.at[pl.ds(0, nd_tile)],
                    den_sp.at[pl.ds(s * nd_tile, nd_tile)])

    plsc.subcore_barrier()

    def piece(base, sz):
        pltpu.sync_copy(dst_hbm.at[pl.ds(base, sz)], dst_v.at[pl.ds(0, sz)])
        a = pltpu.async_copy(ex_hbm.at[pl.ds(base, sz)],
                             ex_v.at[pl.ds(0, sz)], sem)
        b = pltpu.async_copy(scaled_hbm.at[pl.ds(base, sz)],
                             rows_v.at[pl.ds(0, sz)], sem)
        a.wait()
        b.wait()
        pltpu.sync_copy(ex_v.at[pl.ds(0, sz)],
                        den_sp.at[dst_v.at[pl.ds(0, sz)]], add=True)
        pltpu.sync_copy(rows_v.at[pl.ds(0, sz)],
                        agg_sp.at[dst_v.at[pl.ds(0, sz)]], add=True)

    n_full = per_w // CHUNK_S

    def chunk_body(ci, cr):
        piece(base_w + ci * CHUNK_S, CHUNK_S)
        return cr

    lax.fori_loop(0, n_full, chunk_body, jnp.int32(0))
    tail = per_w - n_full * CHUNK_S
    if tail:
        piece(base_w + n_full * CHUNK_S, tail)

    plsc.subcore_barrier()

    def _drain_agg_rows(r0, total):
        done = 0
        while done < total:
            sz = min(CHUNK_S, total - done)
            pltpu.sync_copy(agg_sp.at[pl.ds(r0 + done, sz)],
                            rows_v.at[pl.ds(0, sz)])
            pltpu.sync_copy(rows_v.at[pl.ds(0, sz)],
                            agg_hbm.at[c, pl.ds(r0 + done, sz)])
            done += sz

    @pl.when(s < NS - 1)
    def _():
        _drain_agg_rows(s * n_t0, n_t0)

    @pl.when(s == NS - 1)
    def _():
        _drain_agg_rows((NS - 1) * n_t0, n_last)

    pltpu.sync_copy(den_sp.at[pl.ds(s * nd_tile, nd_tile)],
                    zbuf.at[pl.ds(0, nd_tile)])
    pltpu.sync_copy(zbuf.at[pl.ds(0, nd_tile)],
                    den_hbm.at[pl.ds(c * nd_pad + s * nd_tile, nd_tile)])


def kernel(x, edge_index, edge_attr, batch, W_l, W_r, att, W_e, b_gat,
           W_lin, b_lin):
    n = x.shape[0]
    e = edge_index.shape[1]
    num_graphs = 64
    num_classes = W_lin.shape[1]
    nd_pad = ((n + NS * 8 - 1) // (NS * 8)) * (NS * 8)
    ep = ((e + BM_E - 1) // BM_E) * BM_E

    x_l, x_r = _project(x, W_l, W_r)

    pad = ep - e
    spread = jnp.arange(pad, dtype=jnp.int32) % n
    src = jnp.concatenate([edge_index[0], spread])
    dst_g = jnp.concatenate([edge_index[1], spread])
    dst_s = jnp.concatenate([edge_index[1],
                             n + (jnp.arange(pad, dtype=jnp.int32) % 8)])
    ea = jnp.concatenate([edge_attr[:, 0], jnp.zeros((pad,), jnp.float32)])

    mesh = plsc.VectorSubcoreMesh(core_axis_name="c", subcore_axis_name="s",
                                  num_cores=NC, num_subcores=NS)
    sc_params = pltpu.CompilerParams(needs_layout_passes=False)

    gather_fn = pl.kernel(
        _gather_body,
        out_type=[
            jax.ShapeDtypeStruct((ep, F_OUT), jnp.float32),
            jax.ShapeDtypeStruct((ep, F_OUT), jnp.float32),
        ],
        mesh=mesh,
        compiler_params=sc_params,
        scratch_types=[
            pltpu.VMEM((CHUNK,), jnp.int32),
            pltpu.VMEM((CHUNK,), jnp.int32),
            pltpu.VMEM((CHUNK, F_OUT), jnp.float32),
            pltpu.VMEM((CHUNK, F_OUT), jnp.float32),
            pltpu.SemaphoreType.DMA,
        ],
    )
    xlr, xrr = gather_fn(x_l, x_r, src, dst_g)

    we2 = W_e
    att2 = att[:, None]
    lg, bmax = _edge_logits(xlr, xrr, ea, we2, att2)
    ex, scaled = _edge_scale(lg, xlr, bmax)

    scatter_fn = pl.kernel(
        _scatter_body,
        out_type=[
            jax.ShapeDtypeStruct((NC, n, F_OUT), jnp.float32),
            jax.ShapeDtypeStruct((NC * nd_pad,), jnp.float32),
        ],
        mesh=mesh,
        compiler_params=sc_params,
        scratch_types=[
            pltpu.VMEM((CHUNK_S,), jnp.int32),
            pltpu.VMEM((CHUNK_S,), jnp.float32),
            pltpu.VMEM((CHUNK_S, F_OUT), jnp.float32),
            pltpu.VMEM((640,), jnp.float32),
            pltpu.VMEM_SHARED((n + 8, F_OUT), jnp.float32),
            pltpu.VMEM_SHARED((nd_pad,), jnp.float32),
            pltpu.SemaphoreType.DMA,
        ],
    )
    agg2, den2 = scatter_fn(scaled, ex, dst_s)

    d0 = den2[:n, None]
    d1 = den2[nd_pad:nd_pad + n, None]
    batch2 = batch[:, None]
    bgat2 = b_gat[None, :]
    blin2 = b_lin[None, :]
    return _finish(agg2, d0, d1, batch2, bgat2, W_lin, blin2,
                   num_graphs, num_classes)


# speedup vs baseline: 1.4888x; 1.1145x over previous
import jax
import jax.numpy as jnp
from jax import lax
from jax.experimental import pallas as pl
from jax.experimental.pallas import tpu as pltpu
from jax.experimental.pallas import tpu_sc as plsc

NC = 2
NS = 16
L = 16

F_OUT = 128
CHUNK = 400
CHUNK_S = 256
BM_E = 2048
NEG_SLOPE = 0.2


def _mm_body(x_ref, wl_ref, wr_ref, xl_ref, xr_ref):
    xb = x_ref[...]
    xl_ref[...] = jnp.dot(xb, wl_ref[...], preferred_element_type=jnp.float32)
    xr_ref[...] = jnp.dot(xb, wr_ref[...], preferred_element_type=jnp.float32)


def _project(x, W_l, W_r):
    n, f_in = x.shape
    bm = 1000
    return pl.pallas_call(
        _mm_body,
        grid=(n // bm,),
        in_specs=[
            pl.BlockSpec((bm, f_in), lambda i: (i, 0)),
            pl.BlockSpec((f_in, F_OUT), lambda i: (0, 0)),
            pl.BlockSpec((f_in, F_OUT), lambda i: (0, 0)),
        ],
        out_specs=[
            pl.BlockSpec((bm, F_OUT), lambda i: (i, 0)),
            pl.BlockSpec((bm, F_OUT), lambda i: (i, 0)),
        ],
        out_shape=[
            jax.ShapeDtypeStruct((n, F_OUT), jnp.float32),
            jax.ShapeDtypeStruct((n, F_OUT), jnp.float32),
        ],
    )(x, W_l, W_r)


def _logits_body(xlr_ref, xrr_ref, ea_ref, we_ref, att_ref,
                 lg_ref, bmax_ref, max_acc):
    i = pl.program_id(0)
    nb = pl.num_programs(0)
    ea = ea_ref[...].reshape(BM_E, 1)
    m = xlr_ref[...] + xrr_ref[...] + ea * we_ref[...]
    m = jnp.maximum(m, m * NEG_SLOPE)
    lg = jnp.dot(m, att_ref[...], preferred_element_type=jnp.float32)
    lg_ref[...] = lg.reshape(BM_E)

    @pl.when(i == 0)
    def _():
        max_acc[...] = jnp.full_like(max_acc, -jnp.inf)

    max_acc[...] = jnp.maximum(max_acc[...], jnp.max(lg))

    @pl.when(i == nb - 1)
    def _():
        bmax_ref[...] = max_acc[...]


def _edge_logits(xlr, xrr, ea, we2, att2):
    e = xlr.shape[0]
    nb = e // BM_E
    return pl.pallas_call(
        _logits_body,
        grid=(nb,),
        in_specs=[
            pl.BlockSpec((BM_E, F_OUT), lambda i: (i, 0)),
            pl.BlockSpec((BM_E, F_OUT), lambda i: (i, 0)),
            pl.BlockSpec((BM_E,), lambda i: (i,)),
            pl.BlockSpec((1, F_OUT), lambda i: (0, 0)),
            pl.BlockSpec((F_OUT, 1), lambda i: (0, 0)),
        ],
        out_specs=[
            pl.BlockSpec((BM_E,), lambda i: (i,)),
            pl.BlockSpec((1, F_OUT), lambda i: (0, 0)),
        ],
        out_shape=[
            jax.ShapeDtypeStruct((e,), jnp.float32),
            jax.ShapeDtypeStruct((1, F_OUT), jnp.float32),
        ],
        scratch_shapes=[pltpu.VMEM((1, F_OUT), jnp.float32)],
    )(xlr, xrr, ea, we2, att2)


def _scale_body(lg_ref, xlr_ref, bmax0_ref, bmax1_ref, ex_ref, scaled_ref):
    gmax = jnp.maximum(jnp.max(bmax0_ref[...]), jnp.max(bmax1_ref[...]))
    ex = jnp.exp(lg_ref[...] - gmax)
    ex_ref[...] = ex
    scaled_ref[...] = xlr_ref[...] * ex.reshape(BM_E, 1)


def _edge_scale(lg, xlr, bmax0, bmax1):
    e = xlr.shape[0]
    nb = e // BM_E
    return pl.pallas_call(
        _scale_body,
        grid=(nb,),
        in_specs=[
            pl.BlockSpec((BM_E,), lambda i: (i,)),
            pl.BlockSpec((BM_E, F_OUT), lambda i: (i, 0)),
            pl.BlockSpec((1, F_OUT), lambda i: (0, 0)),
            pl.BlockSpec((1, F_OUT), lambda i: (0, 0)),
        ],
        out_specs=[
            pl.BlockSpec((BM_E,), lambda i: (i,)),
            pl.BlockSpec((BM_E, F_OUT), lambda i: (i, 0)),
        ],
        out_shape=[
            jax.ShapeDtypeStruct((e,), jnp.float32),
            jax.ShapeDtypeStruct((e, F_OUT), jnp.float32),
        ],
    )(lg, xlr, bmax0, bmax1)


def _finish_body(agg_a_ref, agg_b_ref, da0_ref, da1_ref, db0_ref, db1_ref,
                 batch_ref, bgat_ref, wlin_ref,
                 blin_ref, out_ref, pooled_acc, cnt_acc):
    i = pl.program_id(0)
    nb = pl.num_programs(0)
    num_graphs = out_ref.shape[0]

    @pl.when(i == 0)
    def _():
        pooled_acc[...] = jnp.zeros_like(pooled_acc)
        cnt_acc[...] = jnp.zeros_like(cnt_acc)

    den = da0_ref[...] + da1_ref[...] + db0_ref[...] + db1_ref[...] + 1e-16
    h = (agg_a_ref[0] + agg_a_ref[1] + agg_b_ref[0] + agg_b_ref[1]) / den \
        + bgat_ref[...]
    h = jnp.maximum(h, 0.0)

    gids = lax.broadcasted_iota(jnp.int32, (1, num_graphs), 1)
    oneh = (batch_ref[...] == gids).astype(jnp.float32)
    dims = (((0,), (0,)), ((), ()))
    pooled_acc[...] += lax.dot_general(oneh, h, dims,
                                       preferred_element_type=jnp.float32)
    ones = jnp.ones(h.shape, jnp.float32)
    cnt_acc[...] += lax.dot_general(oneh, ones, dims,
                                    preferred_element_type=jnp.float32)

    @pl.when(i == nb - 1)
    def _():
        pooled = pooled_acc[...] / jnp.maximum(cnt_acc[...], 1.0)
        out_ref[...] = jnp.dot(pooled, wlin_ref[...],
                               preferred_element_type=jnp.float32) + blin_ref[...]


def _finish(agg_a, agg_b, dens, batch2, bgat2, W_lin, blin2,
            num_graphs, num_classes):
    n = agg_a.shape[1]
    bm = 1000
    return pl.pallas_call(
        _finish_body,
        grid=(n // bm,),
        in_specs=[
            pl.BlockSpec((NC, bm, F_OUT), lambda i: (0, i, 0)),
            pl.BlockSpec((NC, bm, F_OUT), lambda i: (0, i, 0)),
            pl.BlockSpec((bm, 1), lambda i: (i, 0)),
            pl.BlockSpec((bm, 1), lambda i: (i, 0)),
            pl.BlockSpec((bm, 1), lambda i: (i, 0)),
            pl.BlockSpec((bm, 1), lambda i: (i, 0)),
            pl.BlockSpec((bm, 1), lambda i: (i, 0)),
            pl.BlockSpec((1, F_OUT), lambda i: (0, 0)),
            pl.BlockSpec((F_OUT, num_classes), lambda i: (0, 0)),
            pl.BlockSpec((1, num_classes), lambda i: (0, 0)),
        ],
        out_specs=pl.BlockSpec((num_graphs, num_classes), lambda i: (0, 0)),
        out_shape=jax.ShapeDtypeStruct((num_graphs, num_classes), jnp.float32),
        scratch_shapes=[
            pltpu.VMEM((num_graphs, F_OUT), jnp.float32),
            pltpu.VMEM((num_graphs, F_OUT), jnp.float32),
        ],
    )(agg_a, agg_b, *dens, batch2, bgat2, W_lin, blin2)


def _gather_body(xl_hbm, xr_hbm, src_hbm, dst_hbm, xlr_hbm, xrr_hbm,
                 src_v, dst_v, xl_rows, xr_rows, sem):
    c = lax.axis_index("c")
    s = lax.axis_index("s")
    wid = c * NS + s
    e_total = src_hbm.shape[0]
    per_w = e_total // (NC * NS)
    base_w = wid * per_w

    def piece(base, sz):
        pltpu.sync_copy(src_hbm.at[pl.ds(base, sz)], src_v.at[pl.ds(0, sz)])
        pltpu.sync_copy(dst_hbm.at[pl.ds(base, sz)], dst_v.at[pl.ds(0, sz)])
        a = pltpu.async_copy(xl_hbm.at[src_v.at[pl.ds(0, sz)]],
                             xl_rows.at[pl.ds(0, sz)], sem)
        b = pltpu.async_copy(xr_hbm.at[dst_v.at[pl.ds(0, sz)]],
                             xr_rows.at[pl.ds(0, sz)], sem)
        a.wait()
        b.wait()
        pltpu.sync_copy(xl_rows.at[pl.ds(0, sz)], xlr_hbm.at[pl.ds(base, sz)])
        pltpu.sync_copy(xr_rows.at[pl.ds(0, sz)], xrr_hbm.at[pl.ds(base, sz)])

    n_full = per_w // CHUNK

    def chunk_body(ci, cr):
        piece(base_w + ci * CHUNK, CHUNK)
        return cr

    lax.fori_loop(0, n_full, chunk_body, jnp.int32(0))
    tail = per_w - n_full * CHUNK
    if tail:
        piece(base_w + n_full * CHUNK, tail)


def _scatter_body(scaled_hbm, ex_hbm, dst_hbm,
                  agg_hbm, den_hbm,
                  dst_v, ex_v, rows_v, zbuf,
                  agg_sp, den_sp, sem):
    c = lax.axis_index("c")
    s = lax.axis_index("s")
    wid = c * NS + s
    e_total = dst_hbm.shape[0]
    n = agg_hbm.shape[1]
    per_w = e_total // (NC * NS)
    base_w = wid * per_w

    n_t0 = (n // NS) // 8 * 8
    n_last = n - n_t0 * (NS - 1)
    n_pad_rows = agg_sp.shape[0] - n
    nd_pad = den_sp.shape[0]
    nd_tile = nd_pad // NS

    zv = jnp.zeros((L,), jnp.float32)

    def zrow(r, cr):
        def zcol(k, ck):
            rows_v[r, pl.ds(k * L, L)] = zv
            return ck
        return lax.fori_loop(0, F_OUT // L, zcol, cr)

    lax.fori_loop(0, CHUNK_S, zrow, jnp.int32(0))

    def zflat(k, ck):
        zbuf[pl.ds(k * L, L)] = zv
        return ck

    lax.fori_loop(0, zbuf.shape[0] // L, zflat, jnp.int32(0))

    def _zero_agg_rows(r0, total):
        done = 0
        while total - done > 0:
            sz = min(CHUNK_S, total - done)
            pltpu.sync_copy(rows_v.at[pl.ds(0, sz)],
                            agg_sp.at[pl.ds(r0 + done, sz)])
            done += sz

    @pl.when(s < NS - 1)
    def _():
        _zero_agg_rows(s * n_t0, n_t0)

    @pl.when(s == NS - 1)
    def _():
        _zero_agg_rows((NS - 1) * n_t0, n_last + n_pad_rows)

    pltpu.sync_copy(zbuf.at[pl.ds(0, nd_tile)],
                    den_sp.at[pl.ds(s * nd_tile, nd_tile)])

    plsc.subcore_barrier()

    def piece(base, sz):
        pltpu.sync_copy(dst_hbm.at[pl.ds(base, sz)], dst_v.at[pl.ds(0, sz)])
        a = pltpu.async_copy(ex_hbm.at[pl.ds(base, sz)],
                             ex_v.at[pl.ds(0, sz)], sem)
        b = pltpu.async_copy(scaled_hbm.at[pl.ds(base, sz)],
                             rows_v.at[pl.ds(0, sz)], sem)
        a.wait()
        b.wait()
        pltpu.sync_copy(ex_v.at[pl.ds(0, sz)],
                        den_sp.at[dst_v.at[pl.ds(0, sz)]], add=True)
        pltpu.sync_copy(rows_v.at[pl.ds(0, sz)],
                        agg_sp.at[dst_v.at[pl.ds(0, sz)]], add=True)

    n_full = per_w // CHUNK_S

    def chunk_body(ci, cr):
        piece(base_w + ci * CHUNK_S, CHUNK_S)
        return cr

    lax.fori_loop(0, n_full, chunk_body, jnp.int32(0))
    tail = per_w - n_full * CHUNK_S
    if tail:
        piece(base_w + n_full * CHUNK_S, tail)

    plsc.subcore_barrier()

    def _drain_agg_rows(r0, total):
        done = 0
        while done < total:
            sz = min(CHUNK_S, total - done)
            pltpu.sync_copy(agg_sp.at[pl.ds(r0 + done, sz)],
                            rows_v.at[pl.ds(0, sz)])
            pltpu.sync_copy(rows_v.at[pl.ds(0, sz)],
                            agg_hbm.at[c, pl.ds(r0 + done, sz)])
            done += sz

    @pl.when(s < NS - 1)
    def _():
        _drain_agg_rows(s * n_t0, n_t0)

    @pl.when(s == NS - 1)
    def _():
        _drain_agg_rows((NS - 1) * n_t0, n_last)

    pltpu.sync_copy(den_sp.at[pl.ds(s * nd_tile, nd_tile)],
                    zbuf.at[pl.ds(0, nd_tile)])
    pltpu.sync_copy(zbuf.at[pl.ds(0, nd_tile)],
                    den_hbm.at[pl.ds(c * nd_pad + s * nd_tile, nd_tile)])


def kernel(x, edge_index, edge_attr, batch, W_l, W_r, att, W_e, b_gat,
           W_lin, b_lin):
    n = x.shape[0]
    e = edge_index.shape[1]
    num_graphs = 64
    num_classes = W_lin.shape[1]
    nd_pad = ((n + NS * 8 - 1) // (NS * 8)) * (NS * 8)
    ep = ((e + BM_E - 1) // BM_E) * BM_E

    x_l, x_r = _project(x, W_l, W_r)

    pad = ep - e
    spread = jnp.arange(pad, dtype=jnp.int32) % n
    src = jnp.concatenate([edge_index[0], spread])
    dst_g = jnp.concatenate([edge_index[1], spread])
    dst_s = jnp.concatenate([edge_index[1],
                             n + (jnp.arange(pad, dtype=jnp.int32) % 8)])
    ea = jnp.concatenate([edge_attr[:, 0], jnp.zeros((pad,), jnp.float32)])

    mesh = plsc.VectorSubcoreMesh(core_axis_name="c", subcore_axis_name="s",
                                  num_cores=NC, num_subcores=NS)
    sc_params = pltpu.CompilerParams(needs_layout_passes=False)

    eh0 = (ep // (2 * BM_E)) * BM_E
    halves = [(0, eh0), (eh0, ep - eh0)]

    def gather_fn(src_h, dst_h, xl, xr, eh):
        return pl.kernel(
            _gather_body,
            out_type=[
                jax.ShapeDtypeStruct((eh, F_OUT), jnp.float32),
                jax.ShapeDtypeStruct((eh, F_OUT), jnp.float32),
            ],
            mesh=mesh,
            compiler_params=sc_params,
            scratch_types=[
                pltpu.VMEM((CHUNK,), jnp.int32),
                pltpu.VMEM((CHUNK,), jnp.int32),
                pltpu.VMEM((CHUNK, F_OUT), jnp.float32),
                pltpu.VMEM((CHUNK, F_OUT), jnp.float32),
                pltpu.SemaphoreType.DMA,
            ],
        )(xl, xr, src_h, dst_h)

    def scatter_fn(scaled_h, ex_h, dst_h):
        return pl.kernel(
            _scatter_body,
            out_type=[
                jax.ShapeDtypeStruct((NC, n, F_OUT), jnp.float32),
                jax.ShapeDtypeStruct((NC * nd_pad,), jnp.float32),
            ],
            mesh=mesh,
            compiler_params=sc_params,
            scratch_types=[
                pltpu.VMEM((CHUNK_S,), jnp.int32),
                pltpu.VMEM((CHUNK_S,), jnp.float32),
                pltpu.VMEM((CHUNK_S, F_OUT), jnp.float32),
                pltpu.VMEM((640,), jnp.float32),
                pltpu.VMEM_SHARED((n + 8, F_OUT), jnp.float32),
                pltpu.VMEM_SHARED((nd_pad,), jnp.float32),
                pltpu.SemaphoreType.DMA,
            ],
        )(scaled_h, ex_h, dst_h)

    we2 = W_e
    att2 = att[:, None]

    xlrs, lgs, bmaxs = [], [], []
    for off, eh in halves:
        xlr, xrr = gather_fn(lax.dynamic_slice(src, (off,), (eh,)),
                             lax.dynamic_slice(dst_g, (off,), (eh,)),
                             x_l, x_r, eh)
        lg, bmax = _edge_logits(xlr, xrr,
                                lax.dynamic_slice(ea, (off,), (eh,)),
                                we2, att2)
        xlrs.append(xlr)
        lgs.append(lg)
        bmaxs.append(bmax)

    aggs, dens = [], []
    for (off, eh), xlr, lg in zip(halves, xlrs, lgs):
        ex, scaled = _edge_scale(lg, xlr, bmaxs[0], bmaxs[1])
        agg2, den2 = scatter_fn(scaled, ex,
                                lax.dynamic_slice(dst_s, (off,), (eh,)))
        aggs.append(agg2)
        dens.append(den2[:n, None])
        dens.append(den2[nd_pad:nd_pad + n, None])

    batch2 = batch[:, None]
    bgat2 = b_gat[None, :]
    blin2 = b_lin[None, :]
    return _finish(aggs[0], aggs[1], dens, batch2, bgat2, W_lin, blin2,
                   num_graphs, num_classes)
